# Initial kernel scaffold; baseline (speedup 1.0000x reference)
import jax
import jax.numpy as jnp
from jax.experimental import pallas as pl


def kernel(x, conv_w, conv_b, Wq, bq, keys):
    raise NotImplementedError("write your pallas kernel here")


# speedup vs baseline: 1.5879x; 1.5879x over previous
import functools

import jax
import jax.numpy as jnp
import numpy as np
from jax import lax
from jax.experimental import pallas as pl

KNN = 32


def _candidates():
    pairs = [(r, c) for r in range(KNN) for c in range(KNN)
             if (r + 1) * (c + 1) <= KNN]
    pairs.sort(key=lambda rc: rc[0] * KNN + rc[1])
    n = len(pairs)
    npad = 128
    p = np.full((npad,), 4 * KNN * KNN, np.int32)
    for j, (r, c) in enumerate(pairs):
        p[j] = r * KNN + c
    cnts = [KNN // (r + 1) for r in range(KNN)]
    assert sum(cnts) == n
    return p, cnts, n, npad


_CAND_P, _CAND_CNTS, _NCAND, _NPAD = _candidates()


def _topk32(x, n):
    TB = x.shape[0]
    iota = lax.broadcasted_iota(jnp.int32, (TB, n), 1)
    iota_k = lax.broadcasted_iota(jnp.int32, (TB, KNN), 1)
    neginf = jnp.float32(-jnp.inf)

    def body(k, carry):
        xc, vals, idxs = carry
        m = jnp.max(xc, axis=1, keepdims=True)
        p = jnp.min(jnp.where(xc == m, iota, n), axis=1, keepdims=True)
        xc = jnp.where(iota == p, neginf, xc)
        vals = jnp.where(iota_k == k, m, vals)
        idxs = jnp.where(iota_k == k, p, idxs)
        return xc, vals, idxs

    vals0 = jnp.zeros((TB, KNN), jnp.float32)
    idxs0 = jnp.zeros((TB, KNN), jnp.int32)
    _, vals, idxs = lax.fori_loop(0, KNN, body, (x, vals0, idxs0))
    return vals, idxs


def _combine(s1, i1, s2, i2, key_num, pvec):
    TB = s1.shape[0]
    pconst = jnp.broadcast_to(pvec, (TB, _NPAD))
    neginf = jnp.float32(-jnp.inf)
    pieces = [s1[:, r:r + 1] + s2[:, :cnt]
              for r, cnt in enumerate(_CAND_CNTS) if cnt > 0]
    pieces.append(jnp.full((TB, _NPAD - _NCAND), neginf, jnp.float32))
    cand = jnp.concatenate(pieces, axis=1)
    iota_k = lax.broadcasted_iota(jnp.int32, (TB, KNN), 1)
    sentinel = jnp.int32(4 * KNN * KNN)

    def body(k, carry):
        c_, vals, idxs = carry
        m = jnp.max(c_, axis=1, keepdims=True)
        psel = jnp.min(jnp.where(c_ == m, pconst, sentinel),
                       axis=1, keepdims=True)
        c_ = jnp.where(pconst == psel, neginf, c_)
        r = psel >> 5
        c = psel & (KNN - 1)
        i1sel = jnp.sum(jnp.where(iota_k == r, i1, 0), axis=1, keepdims=True)
        i2sel = jnp.sum(jnp.where(iota_k == c, i2, 0), axis=1, keepdims=True)
        vals = jnp.where(iota_k == k, m, vals)
        idxs = jnp.where(iota_k == k, i1sel * key_num + i2sel, idxs)
        return c_, vals, idxs

    vals0 = jnp.zeros((TB, KNN), jnp.float32)
    idxs0 = jnp.zeros((TB, KNN), jnp.int32)
    _, vals, idxs = lax.fori_loop(0, KNN, body, (cand, vals0, idxs0))
    return vals, idxs


def _pkm_kernel(xb, xm1, xm2, cw, cb, Wq, bq, keys1, keys2, candP,
                scores_out, idx_out, *, heads, key_dim, key_num):
    half = key_dim // 2
    conv = (cb[0][None, :]
            + xm2[...] * cw[0][None, :]
            + xm1[...] * cw[1][None, :]
            + xb[...] * cw[2][None, :])
    query = lax.dot_general(conv, Wq[...], (((1,), (1,)), ((), ())),
                            preferred_element_type=jnp.float32)
    query = query + bq[0][None, :]
    for h in range(heads):
        q1 = query[:, h * key_dim: h * key_dim + half]
        q2 = query[:, h * key_dim + half: (h + 1) * key_dim]
        k1 = keys1[h * key_num:(h + 1) * key_num, :]
        k2 = keys2[h * key_num:(h + 1) * key_num, :]
        sf1 = lax.dot_general(q1, k1, (((1,), (1,)), ((), ())),
                              preferred_element_type=jnp.float32)
        sf2 = lax.dot_general(q2, k2, (((1,), (1,)), ((), ())),
                              preferred_element_type=jnp.float32)
        s1, i1 = _topk32(sf1, key_num)
        s2, i2 = _topk32(sf2, key_num)
        vals, idxs = _combine(s1, i1, s2, i2, key_num, candP[...])
        scores_out[:, h * KNN:(h + 1) * KNN] = vals
        idx_out[:, h * KNN:(h + 1) * KNN] = idxs


def kernel(x, conv_w, conv_b, Wq, bq, keys):
    B, T, C = x.shape
    QD = Wq.shape[0]
    half = keys.shape[1]
    key_dim = 2 * half
    heads = QD // key_dim
    key_num = keys.shape[0] // (2 * heads)
    BT = B * T

    xm1 = jnp.pad(x, ((0, 0), (1, 0), (0, 0)))[:, :T, :].reshape(BT, C)
    xm2 = jnp.pad(x, ((0, 0), (2, 0), (0, 0)))[:, :T, :].reshape(BT, C)
    xf = x.reshape(BT, C)
    cw = conv_w.T
    cb = conv_b[None, :]
    bq2 = bq[None, :]
    keysv = keys.reshape(heads, 2, key_num, half)
    keys1 = keysv[:, 0].reshape(heads * key_num, half)
    keys2 = keysv[:, 1].reshape(heads * key_num, half)

    TB = 256 if BT % 256 == 0 else BT
    grid = (BT // TB,)
    row_spec = pl.BlockSpec((TB, C), lambda i: (i, 0))
    full = lambda shape: pl.BlockSpec(shape, lambda i: tuple(0 for _ in shape))

    scores, indices = pl.pallas_call(
        functools.partial(_pkm_kernel, heads=heads, key_dim=key_dim,
                          key_num=key_num),
        grid=grid,
        in_specs=[
            row_spec, row_spec, row_spec,
            full(cw.shape), full(cb.shape), full(Wq.shape), full(bq2.shape),
            full(keys1.shape), full(keys2.shape),
            full((1, _NPAD)),
        ],
        out_specs=[
            pl.BlockSpec((TB, heads * KNN), lambda i: (i, 0)),
            pl.BlockSpec((TB, heads * KNN), lambda i: (i, 0)),
        ],
        out_shape=[
            jax.ShapeDtypeStruct((BT, heads * KNN), jnp.float32),
            jax.ShapeDtypeStruct((BT, heads * KNN), jnp.int32),
        ],
    )(xf, xm1, xm2, cw, cb, Wq, bq2, keys1, keys2,
      jnp.asarray(_CAND_P)[None, :])

    return scores.reshape(B, T, heads * KNN), indices.reshape(B, T, heads * KNN)


# speedup vs baseline: 4.8614x; 3.0616x over previous
import functools

import jax
import jax.numpy as jnp
import numpy as np
from jax import lax
from jax.experimental import pallas as pl
from jax.experimental.pallas import tpu as pltpu
from jax.experimental.pallas import tpu_sc as plsc

KNN = 32
L = 16
NC, NS = 2, 16
NW = NC * NS


def _scores_kernel(xb, xm1, xm2, cw, cb, Wq, bq, keys1, keys2,
                   s1_out, s2_out, *, heads, key_dim, key_num):
    half = key_dim // 2
    conv = (cb[0][None, :]
            + xm2[...] * cw[0][None, :]
            + xm1[...] * cw[1][None, :]
            + xb[...] * cw[2][None, :])
    query = lax.dot_general(conv, Wq[...], (((1,), (1,)), ((), ())),
                            preferred_element_type=jnp.float32)
    query = query + bq[0][None, :]
    for h in range(heads):
        q1 = query[:, h * key_dim: h * key_dim + half]
        q2 = query[:, h * key_dim + half: (h + 1) * key_dim]
        k1 = keys1[h * key_num:(h + 1) * key_num, :]
        k2 = keys2[h * key_num:(h + 1) * key_num, :]
        s1_out[h, :, :] = lax.dot_general(
            q1, k1, (((1,), (1,)), ((), ())),
            preferred_element_type=jnp.float32)
        s2_out[h, :, :] = lax.dot_general(
            q2, k2, (((1,), (1,)), ((), ())),
            preferred_element_type=jnp.float32)


def _cand_table():
    pairs = [(r, c) for r in range(KNN) for c in range(KNN)
             if (r + 1) * (c + 1) <= KNN]
    pairs.sort(key=lambda rc: rc[0] * KNN + rc[1])
    npad = 128
    tab = np.zeros((4, npad), np.int32)
    tab[2, :] = 4 * KNN * KNN
    for j, (r, c) in enumerate(pairs):
        tab[0, j] = r
        tab[1, j] = c
        tab[2, j] = r * KNN + c
    return tab


_SC_CAND = _cand_table()


def _cmpsel(a, ia, b, ib):
    m = a >= b
    return (jnp.where(m, a, b), jnp.where(m, ia, ib),
            jnp.where(m, b, a), jnp.where(m, ib, ia))


def _sortkv(k, v, descending):
    return plsc.sort_key_val(k, v, descending=descending)


def _top32_merge_init(v0, i0, v1, i1):
    d0k, d0v = _sortkv(v0, i0, True)
    a1k, a1v = _sortkv(v1, i1, False)
    hi, ihi, lo, ilo = _cmpsel(d0k, d0v, a1k, a1v)
    A0, IA0 = _sortkv(hi, ihi, True)
    A1, IA1 = _sortkv(lo, ilo, True)
    return A0, IA0, A1, IA1


def _top32_merge_step(A0, IA0, A1, IA1, vg, ig):
    bk, bv = _sortkv(vg, ig, False)
    hi1, ihi1, _, _ = _cmpsel(A1, IA1, bk, bv)
    h1k, h1v = _sortkv(hi1, ihi1, False)
    hi2, ihi2, lo2, ilo2 = _cmpsel(A0, IA0, h1k, h1v)
    A0, IA0 = _sortkv(hi2, ihi2, True)
    A1, IA1 = _sortkv(lo2, ilo2, True)
    return A0, IA0, A1, IA1


def _row_top32(buf, h, rb, n_chunks):
    iota = lax.broadcasted_iota(jnp.int32, (L,), 0)
    st = _top32_merge_init(buf[h, rb, pl.ds(0, L)], iota,
                           buf[h, rb, pl.ds(L, L)], iota + L)
    for g in range(2, n_chunks):
        st = _top32_merge_step(*st, buf[h, rb, pl.ds(g * L, L)], iota + g * L)
    return st


def _sc_topk_call(scores1, scores2, key_num):
    H, BT, KN = scores1.shape
    assert BT % NW == 0
    tpw = BT // NW
    NB = 8
    assert tpw % NB == 0
    n_chunks = KN // L
    nc_cand = 128 // L

    mesh = plsc.VectorSubcoreMesh(core_axis_name="c", subcore_axis_name="s",
                                  num_cores=NC, num_subcores=NS)

    @functools.partial(
        pl.kernel,
        out_type=[jax.ShapeDtypeStruct((BT, H * KNN), jnp.float32),
                  jax.ShapeDtypeStruct((BT, H * KNN), jnp.int32)],
        mesh=mesh,
        compiler_params=pltpu.CompilerParams(needs_layout_passes=False),
        scratch_types=[
            pltpu.VMEM((H, NB, KN), jnp.float32),
            pltpu.VMEM((H, NB, KN), jnp.float32),
            pltpu.VMEM((4, 128), jnp.int32),
            pltpu.VMEM((KNN,), jnp.float32),
            pltpu.VMEM((KNN,), jnp.int32),
            pltpu.VMEM((KNN,), jnp.float32),
            pltpu.VMEM((KNN,), jnp.int32),
            pltpu.VMEM((NB, H * KNN), jnp.float32),
            pltpu.VMEM((NB, H * KNN), jnp.int32),
        ],
    )
    def k(s1_hbm, s2_hbm, tab_hbm, outs_hbm, outi_hbm,
          b1, b2, tab, s1v, s1i, s2v, s2i, os_, oi_):
        wid = lax.axis_index("c") * NS + lax.axis_index("s")
        tok0 = wid * tpw
        pltpu.sync_copy(tab_hbm, tab)
        neginf = jnp.float32(-jnp.inf)

        def batch_body(bi, _):
            t0 = tok0 + bi * NB
            for h in range(H):
                pltpu.sync_copy(s1_hbm.at[h, pl.ds(t0, NB), :], b1.at[h])
                pltpu.sync_copy(s2_hbm.at[h, pl.ds(t0, NB), :], b2.at[h])

            def row_body(row, _):
                h = row // NB
                rb = lax.rem(row, NB)
                A0, IA0, A1, IA1 = _row_top32(b1, h, rb, n_chunks)
                B0, IB0, B1, IB1 = _row_top32(b2, h, rb, n_chunks)
                s1v[pl.ds(0, L)] = A0
                s1v[pl.ds(L, L)] = A1
                s1i[pl.ds(0, L)] = IA0
                s1i[pl.ds(L, L)] = IA1
                s2v[pl.ds(0, L)] = B0
                s2v[pl.ds(L, L)] = B1
                s2i[pl.ds(0, L)] = IB0
                s2i[pl.ds(L, L)] = IB1

                def cand_chunk(cc):
                    rj = tab[0, pl.ds(cc * L, L)]
                    cj = tab[1, pl.ds(cc * L, L)]
                    pj = tab[2, pl.ds(cc * L, L)]
                    cv = (plsc.load_gather(s1v, [rj])
                          + plsc.load_gather(s2v, [cj]))
                    cv = jnp.where(pj >= KNN * KNN, neginf, cv)
                    return cv, pj

                cv0, pj0 = cand_chunk(0)
                cv1, pj1 = cand_chunk(1)
                st = _top32_merge_init(cv0, pj0, cv1, pj1)
                for cc in range(2, nc_cand):
                    cvg, pjg = cand_chunk(cc)
                    st = _top32_merge_step(*st, cvg, pjg)
                C0, P0, C1, P1 = st
                r0 = P0 >> 5
                c0 = P0 & (KNN - 1)
                r1 = P1 >> 5
                c1 = P1 & (KNN - 1)
                idx0 = (plsc.load_gather(s1i, [r0]) * key_num
                        + plsc.load_gather(s2i, [c0]))
                idx1 = (plsc.load_gather(s1i, [r1]) * key_num
                        + plsc.load_gather(s2i, [c1]))
                col = h * KNN
                os_[rb, pl.ds(col, L)] = C0
                os_[rb, pl.ds(col + L, L)] = C1
                oi_[rb, pl.ds(col, L)] = idx0
                oi_[rb, pl.ds(col + L, L)] = idx1
                return 0

            lax.fori_loop(0, H * NB, row_body, 0)
            pltpu.sync_copy(os_, outs_hbm.at[pl.ds(t0, NB), :])
            pltpu.sync_copy(oi_, outi_hbm.at[pl.ds(t0, NB), :])
            return 0

        lax.fori_loop(0, tpw // NB, batch_body, 0)

    return k(scores1, scores2, jnp.asarray(_SC_CAND))


def kernel(x, conv_w, conv_b, Wq, bq, keys):
    B, T, C = x.shape
    QD = Wq.shape[0]
    half = keys.shape[1]
    key_dim = 2 * half
    heads = QD // key_dim
    key_num = keys.shape[0] // (2 * heads)
    BT = B * T

    xm1 = jnp.pad(x, ((0, 0), (1, 0), (0, 0)))[:, :T, :].reshape(BT, C)
    xm2 = jnp.pad(x, ((0, 0), (2, 0), (0, 0)))[:, :T, :].reshape(BT, C)
    xf = x.reshape(BT, C)
    cw = conv_w.T
    cb = conv_b[None, :]
    bq2 = bq[None, :]
    keysv = keys.reshape(heads, 2, key_num, half)
    keys1 = keysv[:, 0].reshape(heads * key_num, half)
    keys2 = keysv[:, 1].reshape(heads * key_num, half)

    TB = 256 if BT % 256 == 0 else BT
    grid = (BT // TB,)
    row_spec = pl.BlockSpec((TB, C), lambda i: (i, 0))
    full = lambda shape: pl.BlockSpec(shape, lambda i: tuple(0 for _ in shape))

    s1, s2 = pl.pallas_call(
        functools.partial(_scores_kernel, heads=heads, key_dim=key_dim,
                          key_num=key_num),
        grid=grid,
        in_specs=[
            row_spec, row_spec, row_spec,
            full(cw.shape), full(cb.shape), full(Wq.shape), full(bq2.shape),
            full(keys1.shape), full(keys2.shape),
        ],
        out_specs=[
            pl.BlockSpec((heads, TB, key_num), lambda i: (0, i, 0)),
            pl.BlockSpec((heads, TB, key_num), lambda i: (0, i, 0)),
        ],
        out_shape=[
            jax.ShapeDtypeStruct((heads, BT, key_num), jnp.float32),
            jax.ShapeDtypeStruct((heads, BT, key_num), jnp.float32),
        ],
    )(xf, xm1, xm2, cw, cb, Wq, bq2, keys1, keys2)

    scores, indices = _sc_topk_call(s1, s2, key_num)
    return (scores.reshape(B, T, heads * KNN),
            indices.reshape(B, T, heads * KNN))


# speedup vs baseline: 8.3045x; 1.7083x over previous
import functools

import jax
import jax.numpy as jnp
import numpy as np
from jax import lax
from jax.experimental import pallas as pl
from jax.experimental.pallas import tpu as pltpu
from jax.experimental.pallas import tpu_sc as plsc

KNN = 32
L = 16
NC, NS = 2, 16
NW = NC * NS


def _scores_kernel(xb, xm1, xm2, cw, cb, Wq, bq, keys1, keys2,
                   s1_out, s2_out, *, heads, key_dim, key_num):
    half = key_dim // 2
    conv = (cb[0][None, :]
            + xm2[...] * cw[0][None, :]
            + xm1[...] * cw[1][None, :]
            + xb[...] * cw[2][None, :])
    query = lax.dot_general(conv, Wq[...], (((1,), (1,)), ((), ())),
                            preferred_element_type=jnp.float32)
    query = query + bq[0][None, :]
    for h in range(heads):
        q1 = query[:, h * key_dim: h * key_dim + half]
        q2 = query[:, h * key_dim + half: (h + 1) * key_dim]
        k1 = keys1[h * key_num:(h + 1) * key_num, :]
        k2 = keys2[h * key_num:(h + 1) * key_num, :]
        s1_out[h, :, :] = lax.dot_general(
            q1, k1, (((1,), (1,)), ((), ())),
            preferred_element_type=jnp.float32)
        s2_out[h, :, :] = lax.dot_general(
            q2, k2, (((1,), (1,)), ((), ())),
            preferred_element_type=jnp.float32)


def _cand_table():
    pairs = [(r, c) for r in range(KNN) for c in range(KNN)
             if (r + 1) * (c + 1) <= KNN]
    pairs.sort(key=lambda rc: rc[0] * KNN + rc[1])
    npad = 128
    tab = np.zeros((4, npad), np.int32)
    tab[2, :] = 4 * KNN * KNN
    for j, (r, c) in enumerate(pairs):
        tab[0, j] = r
        tab[1, j] = c
        tab[2, j] = r * KNN + c
    return tab


_SC_CAND = _cand_table()


def _cmpsel(a, ia, b, ib):
    m = a >= b
    return (jnp.where(m, a, b), jnp.where(m, ia, ib),
            jnp.where(m, b, a), jnp.where(m, ib, ia))


def _sortkv(k, v, descending):
    return plsc.sort_key_val(k, v, descending=descending)


def _pair_init(c1, i1, c2, i2):
    b1k, b1v = _sortkv(c1, i1, False)
    b2k, b2v = _sortkv(c2, i2, True)
    hiB, ihiB, loB, iloB = _cmpsel(b1k, b1v, b2k, b2v)
    A0, IA0 = _sortkv(hiB, ihiB, True)
    A1, IA1 = _sortkv(loB, iloB, True)
    return A0, IA0, A1, IA1


def _pair_step(A0, IA0, A1, IA1, c1, i1, c2, i2):
    b1k, b1v = _sortkv(c1, i1, False)
    b2k, b2v = _sortkv(c2, i2, True)
    hiB, ihiB, loB, iloB = _cmpsel(b1k, b1v, b2k, b2v)
    Basc0, iB0 = _sortkv(loB, iloB, False)
    Basc1, iB1 = _sortkv(hiB, ihiB, False)
    h0, ih0, _, _ = _cmpsel(A0, IA0, Basc0, iB0)
    h1, ih1, _, _ = _cmpsel(A1, IA1, Basc1, iB1)
    u, iu, lo, ilo = _cmpsel(h0, ih0, h1, ih1)
    A0, IA0 = _sortkv(u, iu, True)
    A1, IA1 = _sortkv(lo, ilo, True)
    return A0, IA0, A1, IA1


def _row_top32(buf, par, h, rb, n_chunks):
    iota = lax.broadcasted_iota(jnp.int32, (L,), 0)
    ld = lambda g: buf[par, h, rb, pl.ds(g * L, L)]
    st = _pair_init(ld(0), iota, ld(1), iota + L)
    for g in range(2, n_chunks, 2):
        st = _pair_step(*st, ld(g), iota + g * L, ld(g + 1), iota + (g + 1) * L)
    return st


def _sc_topk_call(scores1, scores2, key_num):
    H, BT, KN = scores1.shape
    assert BT % NW == 0
    tpw = BT // NW
    NB = 8
    assert tpw % NB == 0
    n_chunks = KN // L
    nc_cand = 128 // L
    nbatches = tpw // NB

    mesh = plsc.VectorSubcoreMesh(core_axis_name="c", subcore_axis_name="s",
                                  num_cores=NC, num_subcores=NS)

    @functools.partial(
        pl.kernel,
        out_type=[jax.ShapeDtypeStruct((BT, H * KNN), jnp.float32),
                  jax.ShapeDtypeStruct((BT, H * KNN), jnp.int32)],
        mesh=mesh,
        compiler_params=pltpu.CompilerParams(needs_layout_passes=False),
        scratch_types=[
            pltpu.VMEM((2, H, NB, KN), jnp.float32),
            pltpu.VMEM((2, H, NB, KN), jnp.float32),
            pltpu.VMEM((4, 128), jnp.int32),
            pltpu.VMEM((2, KNN), jnp.float32),
            pltpu.VMEM((2, KNN), jnp.int32),
            pltpu.VMEM((2, KNN), jnp.float32),
            pltpu.VMEM((2, KNN), jnp.int32),
            pltpu.VMEM((NB, H * KNN), jnp.float32),
            pltpu.VMEM((NB, H * KNN), jnp.int32),
            pltpu.SemaphoreType.DMA,
        ],
    )
    def k(s1_hbm, s2_hbm, tab_hbm, outs_hbm, outi_hbm,
          b1, b2, tab, s1v, s1i, s2v, s2i, os_, oi_, sem):
        wid = lax.axis_index("c") * NS + lax.axis_index("s")
        tok0 = wid * tpw
        pltpu.sync_copy(tab_hbm, tab)
        neginf = jnp.float32(-jnp.inf)

        def copies(par, t0):
            for h in range(H):
                yield pltpu.make_async_copy(
                    s1_hbm.at[h, pl.ds(t0, NB), :], b1.at[par, h], sem)
                yield pltpu.make_async_copy(
                    s2_hbm.at[h, pl.ds(t0, NB), :], b2.at[par, h], sem)

        for c in copies(0, tok0):
            c.start()

        def process_row(par, h, rb, u):
            A0, IA0, A1, IA1 = _row_top32(b1, par, h, rb, n_chunks)
            B0, IB0, B1, IB1 = _row_top32(b2, par, h, rb, n_chunks)
            s1v[u, pl.ds(0, L)] = A0
            s1v[u, pl.ds(L, L)] = A1
            s1i[u, pl.ds(0, L)] = IA0
            s1i[u, pl.ds(L, L)] = IA1
            s2v[u, pl.ds(0, L)] = B0
            s2v[u, pl.ds(L, L)] = B1
            s2i[u, pl.ds(0, L)] = IB0
            s2i[u, pl.ds(L, L)] = IB1

            def cand_chunk(cc):
                rj = tab[0, pl.ds(cc * L, L)]
                cj = tab[1, pl.ds(cc * L, L)]
                pj = tab[2, pl.ds(cc * L, L)]
                cv = (plsc.load_gather(s1v.at[u], [rj])
                      + plsc.load_gather(s2v.at[u], [cj]))
                cv = jnp.where(pj >= KNN * KNN, neginf, cv)
                return cv, pj

            cv0, pj0 = cand_chunk(0)
            cv1, pj1 = cand_chunk(1)
            st = _pair_init(cv0, pj0, cv1, pj1)
            for cc in range(2, nc_cand, 2):
                cva, pja = cand_chunk(cc)
                cvb, pjb = cand_chunk(cc + 1)
                st = _pair_step(*st, cva, pja, cvb, pjb)
            C0, P0, C1, P1 = st
            r0 = P0 >> 5
            c0 = P0 & (KNN - 1)
            r1 = P1 >> 5
            c1 = P1 & (KNN - 1)
            idx0 = (plsc.load_gather(s1i.at[u], [r0]) * key_num
                    + plsc.load_gather(s2i.at[u], [c0]))
            idx1 = (plsc.load_gather(s1i.at[u], [r1]) * key_num
                    + plsc.load_gather(s2i.at[u], [c1]))
            col = h * KNN
            os_[rb, pl.ds(col, L)] = C0
            os_[rb, pl.ds(col + L, L)] = C1
            oi_[rb, pl.ds(col, L)] = idx0
            oi_[rb, pl.ds(col + L, L)] = idx1

        def batch_body(bi, _):
            par = lax.rem(bi, 2)
            t0 = tok0 + bi * NB
            for c in copies(par, t0):
                c.wait()

            @pl.when(bi + 1 < nbatches)
            def _():
                for c in copies(1 - par, t0 + NB):
                    c.start()

            def rows_body(i, _):
                row = 2 * i
                h = row // NB
                rb = lax.rem(row, NB)
                process_row(par, h, rb, 0)
                process_row(par, h, rb + 1, 1)
                return 0

            lax.fori_loop(0, (H * NB) // 2, rows_body, 0)
            pltpu.sync_copy(os_, outs_hbm.at[pl.ds(t0, NB), :])
            pltpu.sync_copy(oi_, outi_hbm.at[pl.ds(t0, NB), :])
            return 0

        lax.fori_loop(0, nbatches, batch_body, 0)

    return k(scores1, scores2, jnp.asarray(_SC_CAND))


def kernel(x, conv_w, conv_b, Wq, bq, keys):
    B, T, C = x.shape
    QD = Wq.shape[0]
    half = keys.shape[1]
    key_dim = 2 * half
    heads = QD // key_dim
    key_num = keys.shape[0] // (2 * heads)
    BT = B * T

    xm1 = jnp.pad(x, ((0, 0), (1, 0), (0, 0)))[:, :T, :].reshape(BT, C)
    xm2 = jnp.pad(x, ((0, 0), (2, 0), (0, 0)))[:, :T, :].reshape(BT, C)
    xf = x.reshape(BT, C)
    cw = conv_w.T
    cb = conv_b[None, :]
    bq2 = bq[None, :]
    keysv = keys.reshape(heads, 2, key_num, half)
    keys1 = keysv[:, 0].reshape(heads * key_num, half)
    keys2 = keysv[:, 1].reshape(heads * key_num, half)

    TB = 256 if BT % 256 == 0 else BT
    grid = (BT // TB,)
    row_spec = pl.BlockSpec((TB, C), lambda i: (i, 0))
    full = lambda shape: pl.BlockSpec(shape, lambda i: tuple(0 for _ in shape))

    s1, s2 = pl.pallas_call(
        functools.partial(_scores_kernel, heads=heads, key_dim=key_dim,
                          key_num=key_num),
        grid=grid,
        in_specs=[
            row_spec, row_spec, row_spec,
            full(cw.shape), full(cb.shape), full(Wq.shape), full(bq2.shape),
            full(keys1.shape), full(keys2.shape),
        ],
        out_specs=[
            pl.BlockSpec((heads, TB, key_num), lambda i: (0, i, 0)),
            pl.BlockSpec((heads, TB, key_num), lambda i: (0, i, 0)),
        ],
        out_shape=[
            jax.ShapeDtypeStruct((heads, BT, key_num), jnp.float32),
            jax.ShapeDtypeStruct((heads, BT, key_num), jnp.float32),
        ],
    )(xf, xm1, xm2, cw, cb, Wq, bq2, keys1, keys2)

    scores, indices = _sc_topk_call(s1, s2, key_num)
    return (scores.reshape(B, T, heads * KNN),
            indices.reshape(B, T, heads * KNN))
